# Initial kernel scaffold; baseline (speedup 1.0000x reference)
import jax
import jax.numpy as jnp
from jax.experimental import pallas as pl


def kernel(feats, edge_index, W1, b1, W2, b2):
    raise NotImplementedError("write your pallas kernel here")


# speedup vs baseline: 2.6324x; 2.6324x over previous
import functools

import jax
import jax.numpy as jnp
from jax import lax
from jax.experimental import pallas as pl
from jax.experimental.pallas import tpu as pltpu
from jax.experimental.pallas import tpu_sc as plsc

N = 10000
E = 320000
D = 128

NC, NS = 2, 16
NW = NC * NS
NA = 10240
EP = 327680
EW = EP // NW
K = 128
CH = EW // K
RPT = NA // NS

_mesh = plsc.VectorSubcoreMesh(core_axis_name="c", subcore_axis_name="s")

_f32 = jnp.float32


@functools.partial(
    pl.kernel,
    out_type=jax.ShapeDtypeStruct((2 * NA, D), _f32),
    mesh=_mesh,
    scratch_types=[
        pltpu.VMEM((K,), jnp.int32),
        pltpu.VMEM((K,), jnp.int32),
        pltpu.VMEM((K, D), _f32),
        pltpu.VMEM_SHARED((NA, D), _f32),
        pltpu.SemaphoreType.DMA,
    ],
    name="sc_agg",
)
def _sc_agg(z_hbm, src_hbm, dst_hbm, zrow_hbm,
            agg, srcv, dstv, rows, acc_sh, sem):
    c = lax.axis_index("c")
    s = lax.axis_index("s")
    row_sl = pl.ds(s * RPT, RPT)
    pltpu.sync_copy(zrow_hbm.at[row_sl], acc_sh.at[row_sl])
    plsc.subcore_barrier()

    base = (c * NS + s) * EW

    def body(i, carry):
        e0 = base + i * K
        pltpu.sync_copy(src_hbm.at[pl.ds(e0, K)], srcv)
        pltpu.sync_copy(dst_hbm.at[pl.ds(e0, K)], dstv)
        pltpu.async_copy(z_hbm.at[srcv], rows, sem).wait()
        pltpu.sync_copy(rows, acc_sh.at[dstv], add=True)
        return carry

    lax.fori_loop(0, CH, body, 0)
    plsc.subcore_barrier()
    pltpu.sync_copy(acc_sh.at[row_sl], agg.at[pl.ds(c * NA + s * RPT, RPT)])


@functools.partial(
    pl.kernel,
    out_type=jax.ShapeDtypeStruct((2 * NA, D), _f32),
    mesh=_mesh,
    scratch_types=[
        pltpu.VMEM((K,), jnp.int32),
        pltpu.VMEM((K, D), _f32),
        pltpu.VMEM_SHARED((NA, D), _f32),
    ],
    name="sc_deg",
)
def _sc_deg(dst_hbm, zrow_hbm, ones_hbm, deg, dstv, onesv, acc_sh):
    c = lax.axis_index("c")
    s = lax.axis_index("s")
    row_sl = pl.ds(s * RPT, RPT)
    pltpu.sync_copy(zrow_hbm.at[row_sl], acc_sh.at[row_sl])
    pltpu.sync_copy(ones_hbm, onesv)
    plsc.subcore_barrier()

    base = (c * NS + s) * EW

    def body(i, carry):
        e0 = base + i * K
        pltpu.sync_copy(dst_hbm.at[pl.ds(e0, K)], dstv)
        pltpu.sync_copy(onesv, acc_sh.at[dstv], add=True)
        return carry

    lax.fori_loop(0, CH, body, 0)
    plsc.subcore_barrier()
    pltpu.sync_copy(acc_sh.at[row_sl], deg.at[pl.ds(c * NA + s * RPT, RPT)])


BR = 1024


def _mm_body(x_ref, w_ref, o_ref):
    o_ref[...] = lax.dot_general(x_ref[...], w_ref[...],
                                 (((1,), (0,)), ((), ())),
                                 preferred_element_type=_f32)


def _tc_matmul(x, w):
    return pl.pallas_call(
        _mm_body,
        grid=(NA // BR,),
        in_specs=[pl.BlockSpec((BR, D), lambda i: (i, 0)),
                  pl.BlockSpec((D, D), lambda i: (0, 0))],
        out_specs=pl.BlockSpec((BR, D), lambda i: (i, 0)),
        out_shape=jax.ShapeDtypeStruct((NA, D), _f32),
    )(x, w)


def _mid_body(a0, a1, z, d0, d1, b, w, o_ref):
    deg = d0[...][:, 0:1] + d1[...][:, 0:1]
    recip = 1.0 / (deg + 1.0)
    h = (a0[...] + a1[...] + z[...]) * recip + b[...]
    h = jnp.maximum(h, 0.0)
    o_ref[...] = lax.dot_general(h, w[...], (((1,), (0,)), ((), ())),
                                 preferred_element_type=_f32)


def _tc_mid(a0, a1, z, d0, d1, b, w):
    rspec = pl.BlockSpec((BR, D), lambda i: (i, 0))
    return pl.pallas_call(
        _mid_body,
        grid=(NA // BR,),
        in_specs=[rspec, rspec, rspec, rspec, rspec,
                  pl.BlockSpec((1, D), lambda i: (0, 0)),
                  pl.BlockSpec((D, D), lambda i: (0, 0))],
        out_specs=rspec,
        out_shape=jax.ShapeDtypeStruct((NA, D), _f32),
    )(a0, a1, z, d0, d1, b, w)


def _out_body(a0, a1, z, d0, d1, b, o_ref):
    deg = d0[...][:, 0:1] + d1[...][:, 0:1]
    recip = 1.0 / (deg + 1.0)
    o_ref[...] = (a0[...] + a1[...] + z[...]) * recip + b[...]


def _tc_out(a0, a1, z, d0, d1, b):
    rspec = pl.BlockSpec((BR, D), lambda i: (i, 0))
    return pl.pallas_call(
        _out_body,
        grid=(NA // BR,),
        in_specs=[rspec, rspec, rspec, rspec, rspec,
                  pl.BlockSpec((1, D), lambda i: (0, 0))],
        out_specs=rspec,
        out_shape=jax.ShapeDtypeStruct((NA, D), _f32),
    )(a0, a1, z, d0, d1, b)


def kernel(feats, edge_index, W1, b1, W2, b2):
    src = edge_index[0].astype(jnp.int32)
    dst = edge_index[1].astype(jnp.int32)
    srcp = jnp.concatenate([src, jnp.zeros((EP - E,), jnp.int32)])
    dstp = jnp.concatenate([dst, jnp.full((EP - E,), NA - 1, jnp.int32)])
    featsp = jnp.pad(feats, ((0, NA - N), (0, 0)))
    zrow = jnp.zeros((NA, D), _f32)
    ones = jnp.ones((K, D), _f32)
    b1r = b1.reshape(1, D)
    b2r = b2.reshape(1, D)

    z1 = _tc_matmul(featsp, W1)
    deg = _sc_deg(dstp, zrow, ones)
    deg0, deg1 = deg[:NA], deg[NA:]
    agg = _sc_agg(z1, srcp, dstp, zrow)
    z2 = _tc_mid(agg[:NA], agg[NA:], z1, deg0, deg1, b1r, W2)
    aggb = _sc_agg(z2, srcp, dstp, zrow)
    out = _tc_out(aggb[:NA], aggb[NA:], z2, deg0, deg1, b2r)
    return out[:N]


# speedup vs baseline: 3.2244x; 1.2249x over previous
import functools

import jax
import jax.numpy as jnp
from jax import lax
from jax.experimental import pallas as pl
from jax.experimental.pallas import tpu as pltpu
from jax.experimental.pallas import tpu_sc as plsc

N = 10000
E = 320000
D = 128

NC, NS = 2, 16
NW = NC * NS
NA = 10240
EP = 327680
EW = EP // NW
K = 128
CH = EW // K
RPT = NA // NS

_mesh = plsc.VectorSubcoreMesh(core_axis_name="c", subcore_axis_name="s")

_f32 = jnp.float32


@functools.partial(
    pl.kernel,
    out_type=jax.ShapeDtypeStruct((2 * NA, D), _f32),
    mesh=_mesh,
    scratch_types=[
        pltpu.VMEM((2, K), jnp.int32),
        pltpu.VMEM((2, K), jnp.int32),
        pltpu.VMEM((2, K, D), _f32),
        pltpu.VMEM_SHARED((NA, D), _f32),
        pltpu.SemaphoreType.DMA,
        pltpu.SemaphoreType.DMA,
    ],
    name="sc_agg",
)
def _sc_agg(z_hbm, src_hbm, dst_hbm, zrow_hbm,
            agg, srcv2, dstv2, rows2, acc_sh, sem0, sem1):
    c = lax.axis_index("c")
    s = lax.axis_index("s")
    wid = c * NS + s
    base = wid * EW
    row_sl = pl.ds(s * RPT, RPT)
    pltpu.sync_copy(zrow_hbm.at[row_sl], acc_sh.at[row_sl])
    plsc.subcore_barrier()

    pltpu.sync_copy(src_hbm.at[pl.ds(base, K)], srcv2.at[0])
    pltpu.sync_copy(dst_hbm.at[pl.ds(base, K)], dstv2.at[0])
    pltpu.async_copy(z_hbm.at[srcv2.at[0]], rows2.at[0], sem0)

    def body(j, carry):
        i0 = 2 * j
        pltpu.sync_copy(src_hbm.at[pl.ds(base + (i0 + 1) * K, K)], srcv2.at[1])
        pltpu.async_copy(z_hbm.at[srcv2.at[1]], rows2.at[1], sem1)
        pltpu.make_async_copy(z_hbm.at[srcv2.at[0]], rows2.at[0], sem0).wait()
        pltpu.sync_copy(rows2.at[0], acc_sh.at[dstv2.at[0]], add=True)
        nxt = lax.min(i0 + 2, CH - 1)
        pltpu.sync_copy(src_hbm.at[pl.ds(base + nxt * K, K)], srcv2.at[0])
        pltpu.sync_copy(dst_hbm.at[pl.ds(base + nxt * K, K)], dstv2.at[0])
        pltpu.async_copy(z_hbm.at[srcv2.at[0]], rows2.at[0], sem0)
        pltpu.sync_copy(dst_hbm.at[pl.ds(base + (i0 + 1) * K, K)], dstv2.at[1])
        pltpu.make_async_copy(z_hbm.at[srcv2.at[1]], rows2.at[1], sem1).wait()
        pltpu.sync_copy(rows2.at[1], acc_sh.at[dstv2.at[1]], add=True)
        return carry

    lax.fori_loop(0, CH // 2, body, 0)
    pltpu.make_async_copy(z_hbm.at[srcv2.at[0]], rows2.at[0], sem0).wait()
    plsc.subcore_barrier()
    pltpu.sync_copy(acc_sh.at[row_sl], agg.at[pl.ds(c * NA + s * RPT, RPT)])


@functools.partial(
    pl.kernel,
    out_type=jax.ShapeDtypeStruct((2 * NA, D), _f32),
    mesh=_mesh,
    scratch_types=[
        pltpu.VMEM((CH, K), jnp.int32),
        pltpu.VMEM((K, D), _f32),
        pltpu.VMEM_SHARED((NA, D), _f32),
    ],
    name="sc_deg",
)
def _sc_deg(dst3_hbm, zrow_hbm, ones_hbm, deg, dstall, onesv, acc_sh):
    c = lax.axis_index("c")
    s = lax.axis_index("s")
    wid = c * NS + s
    row_sl = pl.ds(s * RPT, RPT)
    pltpu.sync_copy(zrow_hbm.at[row_sl], acc_sh.at[row_sl])
    pltpu.sync_copy(ones_hbm, onesv)
    pltpu.sync_copy(dst3_hbm.at[wid], dstall)
    plsc.subcore_barrier()

    def body(i, carry):
        pltpu.sync_copy(onesv, acc_sh.at[dstall.at[i]], add=True)
        return carry

    lax.fori_loop(0, CH, body, 0)
    plsc.subcore_barrier()
    pltpu.sync_copy(acc_sh.at[row_sl], deg.at[pl.ds(c * NA + s * RPT, RPT)])


BR = 1024


def _mm_body(x_ref, w_ref, o_ref):
    o_ref[...] = lax.dot_general(x_ref[...], w_ref[...],
                                 (((1,), (0,)), ((), ())),
                                 preferred_element_type=_f32)


def _tc_matmul(x, w):
    return pl.pallas_call(
        _mm_body,
        grid=(NA // BR,),
        in_specs=[pl.BlockSpec((BR, D), lambda i: (i, 0)),
                  pl.BlockSpec((D, D), lambda i: (0, 0))],
        out_specs=pl.BlockSpec((BR, D), lambda i: (i, 0)),
        out_shape=jax.ShapeDtypeStruct((NA, D), _f32),
    )(x, w)


def _mid_body(a0, a1, z, d0, d1, b, w, o_ref):
    deg = d0[...][:, 0:1] + d1[...][:, 0:1]
    recip = 1.0 / (deg + 1.0)
    h = (a0[...] + a1[...] + z[...]) * recip + b[...]
    h = jnp.maximum(h, 0.0)
    o_ref[...] = lax.dot_general(h, w[...], (((1,), (0,)), ((), ())),
                                 preferred_element_type=_f32)


def _tc_mid(a0, a1, z, d0, d1, b, w):
    rspec = pl.BlockSpec((BR, D), lambda i: (i, 0))
    return pl.pallas_call(
        _mid_body,
        grid=(NA // BR,),
        in_specs=[rspec, rspec, rspec, rspec, rspec,
                  pl.BlockSpec((1, D), lambda i: (0, 0)),
                  pl.BlockSpec((D, D), lambda i: (0, 0))],
        out_specs=rspec,
        out_shape=jax.ShapeDtypeStruct((NA, D), _f32),
    )(a0, a1, z, d0, d1, b, w)


def _out_body(a0, a1, z, d0, d1, b, o_ref):
    deg = d0[...][:, 0:1] + d1[...][:, 0:1]
    recip = 1.0 / (deg + 1.0)
    o_ref[...] = (a0[...] + a1[...] + z[...]) * recip + b[...]


def _tc_out(a0, a1, z, d0, d1, b):
    rspec = pl.BlockSpec((BR, D), lambda i: (i, 0))
    return pl.pallas_call(
        _out_body,
        grid=(NA // BR,),
        in_specs=[rspec, rspec, rspec, rspec, rspec,
                  pl.BlockSpec((1, D), lambda i: (0, 0))],
        out_specs=rspec,
        out_shape=jax.ShapeDtypeStruct((NA, D), _f32),
    )(a0, a1, z, d0, d1, b)


def kernel(feats, edge_index, W1, b1, W2, b2):
    src = edge_index[0].astype(jnp.int32)
    dst = edge_index[1].astype(jnp.int32)
    srcp = jnp.concatenate([src, jnp.zeros((EP - E,), jnp.int32)])
    dstp = jnp.concatenate([dst, jnp.full((EP - E,), NA - 1, jnp.int32)])
    dstp3 = dstp.reshape(NW, CH, K)
    featsp = jnp.pad(feats, ((0, NA - N), (0, 0)))
    zrow = jnp.zeros((NA, D), _f32)
    ones = jnp.ones((K, D), _f32)
    b1r = b1.reshape(1, D)
    b2r = b2.reshape(1, D)

    z1 = _tc_matmul(featsp, W1)
    deg = _sc_deg(dstp3, zrow, ones)
    deg0, deg1 = deg[:NA], deg[NA:]
    agg = _sc_agg(z1, srcp, dstp, zrow)
    z2 = _tc_mid(agg[:NA], agg[NA:], z1, deg0, deg1, b1r, W2)
    aggb = _sc_agg(z2, srcp, dstp, zrow)
    out = _tc_out(aggb[:NA], aggb[NA:], z2, deg0, deg1, b2r)
    return out[:N]


# speedup vs baseline: 3.2294x; 1.0015x over previous
import functools

import jax
import jax.numpy as jnp
from jax import lax
from jax.experimental import pallas as pl
from jax.experimental.pallas import tpu as pltpu
from jax.experimental.pallas import tpu_sc as plsc

N = 10000
E = 320000
D = 128

NC, NS = 2, 16
NW = NC * NS
NA = 10240
EP = 327680
EW = EP // NW
K = 128
CH = EW // K
KA = 80
CHA = EW // KA
RING = 4
IR = 8
RPT = NA // NS

_mesh = plsc.VectorSubcoreMesh(core_axis_name="c", subcore_axis_name="s")

_f32 = jnp.float32


@functools.partial(
    pl.kernel,
    out_type=jax.ShapeDtypeStruct((2 * NA, D), _f32),
    mesh=_mesh,
    scratch_types=[
        pltpu.VMEM((IR, KA), jnp.int32),
        pltpu.VMEM((IR, KA), jnp.int32),
        pltpu.VMEM((RING, KA, D), _f32),
        pltpu.VMEM_SHARED((NA, D), _f32),
        pltpu.SemaphoreType.DMA((RING,)),
        pltpu.SemaphoreType.DMA((IR,)),
    ],
    name="sc_agg",
)
def _sc_agg(z_hbm, src_hbm, dst_hbm, zrow_hbm,
            agg, srcv, dstv, rows, acc_sh, rsem, isem):
    c = lax.axis_index("c")
    s = lax.axis_index("s")
    base = (c * NS + s) * EW
    row_sl = pl.ds(s * RPT, RPT)
    pltpu.sync_copy(zrow_hbm.at[row_sl], acc_sh.at[row_sl])
    plsc.subcore_barrier()

    def esl(chunk):
        return pl.ds(base + chunk * KA, KA)

    def idx_start(chunk, sl):
        pltpu.async_copy(src_hbm.at[esl(chunk)], srcv.at[sl], isem.at[sl])
        pltpu.async_copy(dst_hbm.at[esl(chunk)], dstv.at[sl], isem.at[sl])

    def idx_wait(sl):
        pltpu.make_async_copy(src_hbm.at[esl(0)], srcv.at[sl], isem.at[sl]).wait()
        pltpu.make_async_copy(dst_hbm.at[esl(0)], dstv.at[sl], isem.at[sl]).wait()

    def gather_start(rsl, isl):
        pltpu.async_copy(z_hbm.at[srcv.at[isl]], rows.at[rsl], rsem.at[rsl])

    def gather_wait(rsl, isl):
        pltpu.make_async_copy(z_hbm.at[srcv.at[isl]], rows.at[rsl],
                              rsem.at[rsl]).wait()

    def scatter(rsl, isl):
        pltpu.sync_copy(rows.at[rsl], acc_sh.at[dstv.at[isl]], add=True)

    for sl in range(IR):
        idx_start(sl, sl)
    for i in range(RING - 1):
        idx_wait(i)
        gather_start(i, i)

    def body(j, carry):
        jb = IR * j
        for b in range(IR):
            idx_wait((b + RING - 1) % IR)
            gather_start((b + RING - 1) % RING, (b + RING - 1) % IR)
            gather_wait(b % RING, b % IR)
            scatter(b % RING, b % IR)
            idx_start(jb + b + IR, b % IR)
        return carry

    lax.fori_loop(0, CHA // IR - 1, body, 0)

    for b in range(IR):
        i = CHA - IR + b
        if i + RING - 1 < CHA:
            idx_wait((b + RING - 1) % IR)
            gather_start((b + RING - 1) % RING, (b + RING - 1) % IR)
        else:
            gather_start((b + RING - 1) % RING, (CHA - 1) % IR)
        gather_wait(b % RING, b % IR)
        scatter(b % RING, b % IR)
    for r in range(RING - 1):
        gather_wait((CHA + r) % RING, (CHA - 1) % IR)

    plsc.subcore_barrier()
    pltpu.sync_copy(acc_sh.at[row_sl], agg.at[pl.ds(c * NA + s * RPT, RPT)])


@functools.partial(
    pl.kernel,
    out_type=jax.ShapeDtypeStruct((2 * NA, D), _f32),
    mesh=_mesh,
    scratch_types=[
        pltpu.VMEM((CH, K), jnp.int32),
        pltpu.VMEM((K, D), _f32),
        pltpu.VMEM_SHARED((NA, D), _f32),
    ],
    name="sc_deg",
)
def _sc_deg(dst3_hbm, zrow_hbm, ones_hbm, deg, dstall, onesv, acc_sh):
    c = lax.axis_index("c")
    s = lax.axis_index("s")
    wid = c * NS + s
    row_sl = pl.ds(s * RPT, RPT)
    pltpu.sync_copy(zrow_hbm.at[row_sl], acc_sh.at[row_sl])
    pltpu.sync_copy(ones_hbm, onesv)
    pltpu.sync_copy(dst3_hbm.at[wid], dstall)
    plsc.subcore_barrier()

    def body(i, carry):
        pltpu.sync_copy(onesv, acc_sh.at[dstall.at[i]], add=True)
        return carry

    lax.fori_loop(0, CH, body, 0)
    plsc.subcore_barrier()
    pltpu.sync_copy(acc_sh.at[row_sl], deg.at[pl.ds(c * NA + s * RPT, RPT)])


BR = 1024


def _mm_body(x_ref, w_ref, o_ref):
    o_ref[...] = lax.dot_general(x_ref[...], w_ref[...],
                                 (((1,), (0,)), ((), ())),
                                 preferred_element_type=_f32)


def _tc_matmul(x, w):
    return pl.pallas_call(
        _mm_body,
        grid=(NA // BR,),
        in_specs=[pl.BlockSpec((BR, D), lambda i: (i, 0)),
                  pl.BlockSpec((D, D), lambda i: (0, 0))],
        out_specs=pl.BlockSpec((BR, D), lambda i: (i, 0)),
        out_shape=jax.ShapeDtypeStruct((NA, D), _f32),
    )(x, w)


def _mid_body(a0, a1, z, d0, d1, b, w, o_ref):
    deg = d0[...][:, 0:1] + d1[...][:, 0:1]
    recip = 1.0 / (deg + 1.0)
    h = (a0[...] + a1[...] + z[...]) * recip + b[...]
    h = jnp.maximum(h, 0.0)
    o_ref[...] = lax.dot_general(h, w[...], (((1,), (0,)), ((), ())),
                                 preferred_element_type=_f32)


def _tc_mid(a0, a1, z, d0, d1, b, w):
    rspec = pl.BlockSpec((BR, D), lambda i: (i, 0))
    return pl.pallas_call(
        _mid_body,
        grid=(NA // BR,),
        in_specs=[rspec, rspec, rspec, rspec, rspec,
                  pl.BlockSpec((1, D), lambda i: (0, 0)),
                  pl.BlockSpec((D, D), lambda i: (0, 0))],
        out_specs=rspec,
        out_shape=jax.ShapeDtypeStruct((NA, D), _f32),
    )(a0, a1, z, d0, d1, b, w)


def _out_body(a0, a1, z, d0, d1, b, o_ref):
    deg = d0[...][:, 0:1] + d1[...][:, 0:1]
    recip = 1.0 / (deg + 1.0)
    o_ref[...] = (a0[...] + a1[...] + z[...]) * recip + b[...]


def _tc_out(a0, a1, z, d0, d1, b):
    rspec = pl.BlockSpec((BR, D), lambda i: (i, 0))
    return pl.pallas_call(
        _out_body,
        grid=(NA // BR,),
        in_specs=[rspec, rspec, rspec, rspec, rspec,
                  pl.BlockSpec((1, D), lambda i: (0, 0))],
        out_specs=rspec,
        out_shape=jax.ShapeDtypeStruct((NA, D), _f32),
    )(a0, a1, z, d0, d1, b)


def kernel(feats, edge_index, W1, b1, W2, b2):
    src = edge_index[0].astype(jnp.int32)
    dst = edge_index[1].astype(jnp.int32)
    srcp = jnp.concatenate([src, jnp.zeros((EP - E,), jnp.int32)])
    dstp = jnp.concatenate([dst, jnp.full((EP - E,), NA - 1, jnp.int32)])
    dstp3 = dstp.reshape(NW, CH, K)
    featsp = jnp.pad(feats, ((0, NA - N), (0, 0)))
    zrow = jnp.zeros((NA, D), _f32)
    ones = jnp.ones((K, D), _f32)
    b1r = b1.reshape(1, D)
    b2r = b2.reshape(1, D)

    z1 = _tc_matmul(featsp, W1)
    deg = _sc_deg(dstp3, zrow, ones)
    deg0, deg1 = deg[:NA], deg[NA:]
    agg = _sc_agg(z1, srcp, dstp, zrow)
    z2 = _tc_mid(agg[:NA], agg[NA:], z1, deg0, deg1, b1r, W2)
    aggb = _sc_agg(z2, srcp, dstp, zrow)
    out = _tc_out(aggb[:NA], aggb[NA:], z2, deg0, deg1, b2r)
    return out[:N]


# speedup vs baseline: 3.2687x; 1.0122x over previous
import functools

import jax
import jax.numpy as jnp
from jax import lax
from jax.experimental import pallas as pl
from jax.experimental.pallas import tpu as pltpu
from jax.experimental.pallas import tpu_sc as plsc

N = 10000
E = 320000
D = 128

NC, NS = 2, 16
NW = NC * NS
NA = 10240
EP = 327680
EW = EP // NW
K = 128
CH = EW // K
KA = 80
EW0 = 15360
EW1 = 5120
CHA0 = EW0 // KA
CHA1 = EW1 // KA
RING = 4
IR = 8
RPT = NA // NS

_mesh = plsc.VectorSubcoreMesh(core_axis_name="c", subcore_axis_name="s")

_f32 = jnp.float32


@functools.partial(
    pl.kernel,
    out_type=jax.ShapeDtypeStruct((2 * NA, D), _f32),
    mesh=_mesh,
    scratch_types=[
        pltpu.VMEM((IR, KA), jnp.int32),
        pltpu.VMEM((IR, KA), jnp.int32),
        pltpu.VMEM((RING, KA, D), _f32),
        pltpu.VMEM_SHARED((NA, D), _f32),
        pltpu.SemaphoreType.DMA((RING,)),
        pltpu.SemaphoreType.DMA((IR,)),
    ],
    name="sc_agg",
)
def _sc_agg(z_hbm, src_hbm, dst_hbm, zrow_hbm,
            agg, srcv, dstv, rows, acc_sh, rsem, isem):
    c = lax.axis_index("c")
    s = lax.axis_index("s")
    base = jnp.where(c == 0, s * EW0, NS * EW0 + s * EW1)
    trip = jnp.where(c == 0, CHA0 // IR - 1, CHA1 // IR - 1)
    row_sl = pl.ds(s * RPT, RPT)
    pltpu.sync_copy(zrow_hbm.at[row_sl], acc_sh.at[row_sl])
    plsc.subcore_barrier()

    def esl(chunk):
        return pl.ds(base + chunk * KA, KA)

    def idx_start(chunk, sl):
        pltpu.async_copy(src_hbm.at[esl(chunk)], srcv.at[sl], isem.at[sl])
        pltpu.async_copy(dst_hbm.at[esl(chunk)], dstv.at[sl], isem.at[sl])

    def idx_wait(sl):
        pltpu.make_async_copy(src_hbm.at[esl(0)], srcv.at[sl], isem.at[sl]).wait()
        pltpu.make_async_copy(dst_hbm.at[esl(0)], dstv.at[sl], isem.at[sl]).wait()

    def gather_start(rsl, isl):
        pltpu.async_copy(z_hbm.at[srcv.at[isl]], rows.at[rsl], rsem.at[rsl])

    def gather_wait(rsl, isl):
        pltpu.make_async_copy(z_hbm.at[srcv.at[isl]], rows.at[rsl],
                              rsem.at[rsl]).wait()

    def scatter(rsl, isl):
        pltpu.sync_copy(rows.at[rsl], acc_sh.at[dstv.at[isl]], add=True)

    for sl in range(IR):
        idx_start(sl, sl)
    for i in range(RING - 1):
        idx_wait(i)
        gather_start(i, i)

    def body(j, carry):
        jb = IR * j
        for b in range(IR):
            idx_wait((b + RING - 1) % IR)
            gather_start((b + RING - 1) % RING, (b + RING - 1) % IR)
            gather_wait(b % RING, b % IR)
            scatter(b % RING, b % IR)
            idx_start(jb + b + IR, b % IR)
        return carry

    lax.fori_loop(0, trip, body, 0)

    for b in range(IR):
        if b + RING - 1 < IR:
            idx_wait((b + RING - 1) % IR)
            gather_start((b + RING - 1) % RING, (b + RING - 1) % IR)
        else:
            gather_start((b + RING - 1) % RING, IR - 1)
        gather_wait(b % RING, b % IR)
        scatter(b % RING, b % IR)
    for r in range(RING - 1):
        gather_wait(r % RING, IR - 1)

    plsc.subcore_barrier()
    pltpu.sync_copy(acc_sh.at[row_sl], agg.at[pl.ds(c * NA + s * RPT, RPT)])


@functools.partial(
    pl.kernel,
    out_type=jax.ShapeDtypeStruct((2 * NA, D), _f32),
    mesh=_mesh,
    scratch_types=[
        pltpu.VMEM((CH, K), jnp.int32),
        pltpu.VMEM((K, D), _f32),
        pltpu.VMEM_SHARED((NA, D), _f32),
    ],
    name="sc_deg",
)
def _sc_deg(dst3_hbm, zrow_hbm, ones_hbm, deg, dstall, onesv, acc_sh):
    c = lax.axis_index("c")
    s = lax.axis_index("s")
    wid = c * NS + s
    row_sl = pl.ds(s * RPT, RPT)
    pltpu.sync_copy(zrow_hbm.at[row_sl], acc_sh.at[row_sl])
    pltpu.sync_copy(ones_hbm, onesv)
    pltpu.sync_copy(dst3_hbm.at[wid], dstall)
    plsc.subcore_barrier()

    def body(i, carry):
        pltpu.sync_copy(onesv, acc_sh.at[dstall.at[i]], add=True)
        return carry

    lax.fori_loop(0, CH, body, 0)
    plsc.subcore_barrier()
    pltpu.sync_copy(acc_sh.at[row_sl], deg.at[pl.ds(c * NA + s * RPT, RPT)])


BR = 1024


def _mm_body(x_ref, w_ref, o_ref):
    o_ref[...] = lax.dot_general(x_ref[...], w_ref[...],
                                 (((1,), (0,)), ((), ())),
                                 preferred_element_type=_f32)


def _tc_matmul(x, w):
    return pl.pallas_call(
        _mm_body,
        grid=(NA // BR,),
        in_specs=[pl.BlockSpec((BR, D), lambda i: (i, 0)),
                  pl.BlockSpec((D, D), lambda i: (0, 0))],
        out_specs=pl.BlockSpec((BR, D), lambda i: (i, 0)),
        out_shape=jax.ShapeDtypeStruct((NA, D), _f32),
    )(x, w)


def _mid_body(a0, a1, z, d0, d1, b, w, o_ref):
    deg = d0[...][:, 0:1] + d1[...][:, 0:1]
    recip = 1.0 / (deg + 1.0)
    h = (a0[...] + a1[...] + z[...]) * recip + b[...]
    h = jnp.maximum(h, 0.0)
    o_ref[...] = lax.dot_general(h, w[...], (((1,), (0,)), ((), ())),
                                 preferred_element_type=_f32)


def _tc_mid(a0, a1, z, d0, d1, b, w):
    rspec = pl.BlockSpec((BR, D), lambda i: (i, 0))
    return pl.pallas_call(
        _mid_body,
        grid=(NA // BR,),
        in_specs=[rspec, rspec, rspec, rspec, rspec,
                  pl.BlockSpec((1, D), lambda i: (0, 0)),
                  pl.BlockSpec((D, D), lambda i: (0, 0))],
        out_specs=rspec,
        out_shape=jax.ShapeDtypeStruct((NA, D), _f32),
    )(a0, a1, z, d0, d1, b, w)


def _out_body(a0, a1, z, d0, d1, b, o_ref):
    deg = d0[...][:, 0:1] + d1[...][:, 0:1]
    recip = 1.0 / (deg + 1.0)
    o_ref[...] = (a0[...] + a1[...] + z[...]) * recip + b[...]


def _tc_out(a0, a1, z, d0, d1, b):
    rspec = pl.BlockSpec((BR, D), lambda i: (i, 0))
    return pl.pallas_call(
        _out_body,
        grid=(NA // BR,),
        in_specs=[rspec, rspec, rspec, rspec, rspec,
                  pl.BlockSpec((1, D), lambda i: (0, 0))],
        out_specs=rspec,
        out_shape=jax.ShapeDtypeStruct((NA, D), _f32),
    )(a0, a1, z, d0, d1, b)


def kernel(feats, edge_index, W1, b1, W2, b2):
    src = edge_index[0].astype(jnp.int32)
    dst = edge_index[1].astype(jnp.int32)
    srcp = jnp.concatenate([src, jnp.zeros((EP - E,), jnp.int32)])
    dstp = jnp.concatenate([dst, jnp.full((EP - E,), NA - 1, jnp.int32)])
    dstp3 = dstp.reshape(NW, CH, K)
    featsp = jnp.pad(feats, ((0, NA - N), (0, 0)))
    zrow = jnp.zeros((NA, D), _f32)
    ones = jnp.ones((K, D), _f32)
    b1r = b1.reshape(1, D)
    b2r = b2.reshape(1, D)

    z1 = _tc_matmul(featsp, W1)
    deg = _sc_deg(dstp3, zrow, ones)
    deg0, deg1 = deg[:NA], deg[NA:]
    agg = _sc_agg(z1, srcp, dstp, zrow)
    z2 = _tc_mid(agg[:NA], agg[NA:], z1, deg0, deg1, b1r, W2)
    aggb = _sc_agg(z2, srcp, dstp, zrow)
    out = _tc_out(aggb[:NA], aggb[NA:], z2, deg0, deg1, b2r)
    return out[:N]


# speedup vs baseline: 3.5336x; 1.0811x over previous
import functools

import jax
import jax.numpy as jnp
from jax import lax
from jax.experimental import pallas as pl
from jax.experimental.pallas import tpu as pltpu
from jax.experimental.pallas import tpu_sc as plsc

N = 10000
E = 320000
D = 128

NC, NS = 2, 16
NW = NC * NS
NA = 10240
NASP = 10112
EP = 327680
ET = EP // NS
KA = 64
CHA = ET // KA
KD = 80
CHD = ET // KD
RING = 4
IR = 8
RPT = NASP // NS


_mesh = plsc.VectorSubcoreMesh(core_axis_name="c", subcore_axis_name="s")

_f32 = jnp.float32


@functools.partial(
    pl.kernel,
    out_type=jax.ShapeDtypeStruct((2 * NA, D), _f32),
    mesh=_mesh,
    scratch_types=[
        pltpu.VMEM((IR, KA), jnp.int32),
        pltpu.VMEM((IR, KA), jnp.int32),
        pltpu.VMEM((RING, KA, D), _f32),
        pltpu.VMEM((IR, KD), jnp.int32),
        pltpu.VMEM((KD, D), _f32),
        pltpu.VMEM_SHARED((NASP, D), _f32),
        pltpu.SemaphoreType.DMA((RING,)),
        pltpu.SemaphoreType.DMA((IR,)),
        pltpu.SemaphoreType.DMA((IR,)),
    ],
    name="sc_agg_deg",
)
def _sc_agg_deg(z_hbm, src_hbm, dst_hbm, zrow_hbm, ones_hbm,
                out, srcv, dstv, rows, degv, onesv, acc_sh, rsem, isem, dsem):
    c = lax.axis_index("c")
    s = lax.axis_index("s")
    base = s * ET
    row_sl = pl.ds(s * RPT, RPT)
    pltpu.sync_copy(zrow_hbm.at[row_sl], acc_sh.at[row_sl])
    plsc.subcore_barrier()

    @pl.when(c == 0)
    def _agg():
        def esl(chunk):
            return pl.ds(base + chunk * KA, KA)

        def idx_start(chunk, sl):
            pltpu.async_copy(src_hbm.at[esl(chunk)], srcv.at[sl], isem.at[sl])
            pltpu.async_copy(dst_hbm.at[esl(chunk)], dstv.at[sl], isem.at[sl])

        def idx_wait(sl):
            pltpu.make_async_copy(src_hbm.at[esl(0)], srcv.at[sl],
                                  isem.at[sl]).wait()
            pltpu.make_async_copy(dst_hbm.at[esl(0)], dstv.at[sl],
                                  isem.at[sl]).wait()

        def gather_start(rsl, isl):
            pltpu.async_copy(z_hbm.at[srcv.at[isl]], rows.at[rsl], rsem.at[rsl])

        def gather_wait(rsl, isl):
            pltpu.make_async_copy(z_hbm.at[srcv.at[isl]], rows.at[rsl],
                                  rsem.at[rsl]).wait()

        def scatter(rsl, isl):
            pltpu.sync_copy(rows.at[rsl], acc_sh.at[dstv.at[isl]], add=True)

        for sl in range(IR):
            idx_start(sl, sl)
        for i in range(RING - 1):
            idx_wait(i)
            gather_start(i, i)

        def body(j, carry):
            jb = IR * j
            for b in range(IR):
                idx_wait((b + RING - 1) % IR)
                gather_start((b + RING - 1) % RING, (b + RING - 1) % IR)
                gather_wait(b % RING, b % IR)
                scatter(b % RING, b % IR)
                idx_start(jb + b + IR, b % IR)
            return carry

        lax.fori_loop(0, CHA // IR - 1, body, 0)

        for b in range(IR):
            if b + RING - 1 < IR:
                idx_wait((b + RING - 1) % IR)
                gather_start((b + RING - 1) % RING, (b + RING - 1) % IR)
            else:
                gather_start((b + RING - 1) % RING, IR - 1)
            gather_wait(b % RING, b % IR)
            scatter(b % RING, b % IR)
        for r in range(RING - 1):
            gather_wait(r % RING, IR - 1)

    @pl.when(c == 1)
    def _deg():
        pltpu.sync_copy(ones_hbm, onesv)

        def dsl(chunk):
            return pl.ds(base + chunk * KD, KD)

        def didx_start(chunk, sl):
            pltpu.async_copy(dst_hbm.at[dsl(chunk)], degv.at[sl], dsem.at[sl])

        def didx_wait(sl):
            pltpu.make_async_copy(dst_hbm.at[dsl(0)], degv.at[sl],
                                  dsem.at[sl]).wait()

        for sl in range(IR):
            didx_start(sl, sl)

        def body(j, carry):
            jb = IR * j
            for b in range(IR):
                didx_wait(b)
                pltpu.sync_copy(onesv, acc_sh.at[degv.at[b]], add=True)
                didx_start(jb + b + IR, b)
            return carry

        lax.fori_loop(0, CHD // IR - 1, body, 0)
        for b in range(IR):
            didx_wait(b)
            pltpu.sync_copy(onesv, acc_sh.at[degv.at[b]], add=True)

    plsc.subcore_barrier()
    pltpu.sync_copy(acc_sh.at[row_sl], out.at[pl.ds(c * NA + s * RPT, RPT)])


BR = 1024


def _mm_body(x_ref, w_ref, o_ref):
    o_ref[...] = lax.dot_general(x_ref[...], w_ref[...],
                                 (((1,), (0,)), ((), ())),
                                 preferred_element_type=_f32)


def _tc_matmul(x, w):
    return pl.pallas_call(
        _mm_body,
        grid=(NA // BR,),
        in_specs=[pl.BlockSpec((BR, D), lambda i: (i, 0)),
                  pl.BlockSpec((D, D), lambda i: (0, 0))],
        out_specs=pl.BlockSpec((BR, D), lambda i: (i, 0)),
        out_shape=jax.ShapeDtypeStruct((NA, D), _f32),
    )(x, w)


def _mid_body(a, z, dg, b, w, o_ref):
    recip = 1.0 / (dg[...][:, 0:1] + 1.0)
    h = (a[...] + z[...]) * recip + b[...]
    h = jnp.maximum(h, 0.0)
    o_ref[...] = lax.dot_general(h, w[...], (((1,), (0,)), ((), ())),
                                 preferred_element_type=_f32)


def _tc_mid(a, z, dg, b, w):
    rspec = pl.BlockSpec((BR, D), lambda i: (i, 0))
    return pl.pallas_call(
        _mid_body,
        grid=(NA // BR,),
        in_specs=[rspec, rspec, rspec,
                  pl.BlockSpec((1, D), lambda i: (0, 0)),
                  pl.BlockSpec((D, D), lambda i: (0, 0))],
        out_specs=rspec,
        out_shape=jax.ShapeDtypeStruct((NA, D), _f32),
    )(a, z, dg, b, w)


def _out_body(a, z, dg, b, o_ref):
    recip = 1.0 / (dg[...][:, 0:1] + 1.0)
    o_ref[...] = (a[...] + z[...]) * recip + b[...]


def _tc_out(a, z, dg, b):
    rspec = pl.BlockSpec((BR, D), lambda i: (i, 0))
    return pl.pallas_call(
        _out_body,
        grid=(NA // BR,),
        in_specs=[rspec, rspec, rspec,
                  pl.BlockSpec((1, D), lambda i: (0, 0))],
        out_specs=rspec,
        out_shape=jax.ShapeDtypeStruct((NA, D), _f32),
    )(a, z, dg, b)


def kernel(feats, edge_index, W1, b1, W2, b2):
    src = edge_index[0].astype(jnp.int32)
    dst = edge_index[1].astype(jnp.int32)
    srcp = jnp.concatenate([src, jnp.zeros((EP - E,), jnp.int32)])
    dstp = jnp.concatenate([dst, jnp.full((EP - E,), NASP - 1, jnp.int32)])
    featsp = jnp.pad(feats, ((0, NA - N), (0, 0)))
    zrow = jnp.zeros((NA, D), _f32)
    ones = jnp.ones((KD, D), _f32)
    b1r = b1.reshape(1, D)
    b2r = b2.reshape(1, D)

    z1 = _tc_matmul(featsp, W1)
    o1 = _sc_agg_deg(z1, srcp, dstp, zrow, ones)
    agg1, deg = o1[:NA], o1[NA:]
    z2 = _tc_mid(agg1, z1, deg, b1r, W2)
    o2 = _sc_agg_deg(z2, srcp, dstp, zrow, ones)
    out = _tc_out(o2[:NA], z2, deg, b2r)
    return out[:N]
